# baseline (speedup 1.0000x reference)
import functools

import jax
import jax.numpy as jnp
from jax import lax
from jax.experimental import pallas as pl
from jax.experimental.pallas import tpu as pltpu
from jax.experimental.pallas import tpu_sc as plsc

N = 10000
NPAD = 10240
D = 256
E = 160000
BATCH = 4096
NSC = 2
NTILE = 16
LANES = 16
EPT = E // (NSC * NTILE)
NCHUNK = (EPT + LANES - 1) // LANES
ALPHA = 0.2
LAMBDA_2 = 1e-4

_mesh = plsc.VectorSubcoreMesh(core_axis_name="c", subcore_axis_name="s")


def _dense_gat_body(h_ref, w_ref, a1_ref, a2_ref, wh_ref, s1_ref, s2_ref,
                    ssq_ref):
    i = pl.program_id(0)
    h = h_ref[...]
    wh = jnp.dot(h, w_ref[...], preferred_element_type=jnp.float32)
    wh_ref[...] = wh
    s1_ref[...] = jnp.dot(wh, a1_ref[...], preferred_element_type=jnp.float32)
    s2_ref[...] = jnp.dot(wh, a2_ref[...], preferred_element_type=jnp.float32)

    @pl.when(i == 0)
    def _():
        ssq_ref[...] = jnp.zeros_like(ssq_ref)

    ssq_ref[...] += jnp.sum(h * h)


def _dense_gat(h, w, a):
    bm = 1000
    a1 = a[:D][:, None]
    a2 = a[D:][:, None]
    return pl.pallas_call(
        _dense_gat_body,
        grid=(N // bm,),
        in_specs=[
            pl.BlockSpec((bm, D), lambda i: (i, 0)),
            pl.BlockSpec((D, D), lambda i: (0, 0)),
            pl.BlockSpec((D, 1), lambda i: (0, 0)),
            pl.BlockSpec((D, 1), lambda i: (0, 0)),
        ],
        out_specs=[
            pl.BlockSpec((bm, D), lambda i: (i, 0)),
            pl.BlockSpec((bm, 1), lambda i: (i, 0)),
            pl.BlockSpec((bm, 1), lambda i: (i, 0)),
            pl.BlockSpec((1, 1), lambda i: (0, 0)),
        ],
        out_shape=[
            jax.ShapeDtypeStruct((N, D), jnp.float32),
            jax.ShapeDtypeStruct((N, 1), jnp.float32),
            jax.ShapeDtypeStruct((N, 1), jnp.float32),
            jax.ShapeDtypeStruct((1, 1), jnp.float32),
        ],
    )(h, w, a1, a2)


def _edge_score_body(src_hbm, dst_hbm, s1_hbm, s2_hbm, eexp_hbm, dpart_hbm,
                     s1_tab, s2_tab, src_v, dst_v, ee_v, dloc):
    c = lax.axis_index("c")
    s = lax.axis_index("s")
    wid = c * NTILE + s
    base = wid * EPT
    pltpu.sync_copy(s1_hbm, s1_tab)
    pltpu.sync_copy(s2_hbm, s2_tab)
    pltpu.sync_copy(src_hbm.at[pl.ds(base, EPT)], src_v.at[pl.ds(0, EPT)])
    pltpu.sync_copy(dst_hbm.at[pl.ds(base, EPT)], dst_v.at[pl.ds(0, EPT)])

    lane = lax.iota(jnp.int32, LANES)
    zv = jnp.zeros((LANES,), jnp.float32)

    def zero_body(i, _):
        dloc[pl.ds(i * LANES, LANES)] = zv
        return 0

    lax.fori_loop(0, NPAD // LANES, zero_body, 0)

    def body(i, _):
        m = (i * LANES + lane) < EPT
        si = jnp.where(m, src_v[pl.ds(i * LANES, LANES)], 0)
        di = jnp.where(m, dst_v[pl.ds(i * LANES, LANES)], 0)
        ev = zv
        for r in range(LANES):
            sr = si[r]
            dr = di[r]
            row1 = s1_tab[pl.ds((sr >> 4) * LANES, LANES)]
            v1 = row1[jnp.broadcast_to(sr & 15, (LANES,))]
            row2 = s2_tab[pl.ds((dr >> 4) * LANES, LANES)]
            v2 = row2[jnp.broadcast_to(dr & 15, (LANES,))]
            ev = ev + jnp.where(lane == r, v1 + v2, 0.0)
        ev = jnp.where(ev > 0, ev, ALPHA * ev)
        ev = jnp.exp(ev)
        ee_v[pl.ds(i * LANES, LANES)] = ev
        eem = jnp.where(m, ev, 0.0)
        for r in range(LANES):
            dr = di[r]
            off = (dr >> 4) * LANES
            rowd = dloc[pl.ds(off, LANES)]
            dloc[pl.ds(off, LANES)] = rowd + jnp.where(lane == (dr & 15),
                                                       eem[r], 0.0)
        return 0

    lax.fori_loop(0, NCHUNK, body, 0)
    pltpu.sync_copy(ee_v.at[pl.ds(0, EPT)], eexp_hbm.at[pl.ds(base, EPT)])
    pltpu.sync_copy(dloc, dpart_hbm.at[wid])


def _edge_scores(src, dst, s1, s2):
    pad_e = NCHUNK * LANES
    return pl.kernel(
        _edge_score_body,
        out_type=[
            jax.ShapeDtypeStruct((E,), jnp.float32),
            jax.ShapeDtypeStruct((NSC * NTILE, NPAD), jnp.float32),
        ],
        mesh=_mesh,
        scratch_types=[
            pltpu.VMEM((N,), jnp.float32),
            pltpu.VMEM((N,), jnp.float32),
            pltpu.VMEM((pad_e,), jnp.int32),
            pltpu.VMEM((pad_e,), jnp.int32),
            pltpu.VMEM((pad_e,), jnp.float32),
            pltpu.VMEM((NPAD,), jnp.float32),
        ],
    )(src, dst, s1, s2)


NBUF = 6


def _row_pass_body(nfilt, src_hbm, dst_hbm, w_hbm, tab_hbm, *rest):
    if nfilt:
        ids_hbm, out_hbm = rest[0], rest[1]
        (src_v, dst_v, w_v, idx_g, idx_s, rows, zb,
         ids_v, flag, csrc, cdst, cw) = rest[2:14]
        sems = rest[14:]
    else:
        out_hbm = rest[0]
        (src_v, dst_v, w_v, idx_g, idx_s, rows, zb) = rest[1:8]
        sems = rest[8:]
    gsem = sems[:NBUF]
    ssem = sems[NBUF:]
    c = lax.axis_index("c")
    s = lax.axis_index("s")
    wid = c * NTILE + s
    base = wid * EPT

    z = jnp.zeros((LANES,), jnp.float32)
    for r in range(LANES):
        for cc in range(D // LANES):
            zb[r, pl.ds(cc * LANES, LANES)] = z

    nz = NPAD // NTILE // LANES

    def zloop(j, _):
        pltpu.sync_copy(zb, out_hbm.at[c].at[pl.ds((s * nz + j) * LANES,
                                                   LANES)])
        return 0

    lax.fori_loop(0, nz, zloop, 0)
    plsc.subcore_barrier()

    pltpu.sync_copy(src_hbm.at[pl.ds(base, EPT)], src_v.at[pl.ds(0, EPT)])
    pltpu.sync_copy(dst_hbm.at[pl.ds(base, EPT)], dst_v.at[pl.ds(0, EPT)])
    pltpu.sync_copy(w_hbm.at[pl.ds(base, EPT)], w_v.at[pl.ds(0, EPT)])

    lane = lax.iota(jnp.int32, LANES)

    if nfilt:
        pltpu.sync_copy(ids_hbm, ids_v)
        zv = jnp.zeros((LANES,), jnp.float32)

        def zf(j, _):
            flag[pl.ds(j * LANES, LANES)] = zv
            return 0

        lax.fori_loop(0, NPAD // LANES, zf, 0)

        def fb(j, _):
            idc = ids_v[pl.ds(j * LANES, LANES)]
            for r in range(LANES):
                idr = idc[r]
                off = (idr >> 4) * LANES
                rowf = flag[pl.ds(off, LANES)]
                flag[pl.ds(off, LANES)] = jnp.where(lane == (idr & 15),
                                                    1.0, rowf)
            return 0

        lax.fori_loop(0, nfilt // LANES, fb, 0)

        def cb(j, cnt):
            m1 = jnp.where((j * LANES + lane) < EPT, 1.0, 0.0)
            si = src_v[pl.ds(j * LANES, LANES)]
            di = jnp.where(m1 > 0, dst_v[pl.ds(j * LANES, LANES)], 0)
            wc = w_v[pl.ds(j * LANES, LANES)]
            for r in range(LANES):
                dr = di[r]
                flrow = flag[pl.ds((dr >> 4) * LANES, LANES)]
                fl = flrow[jnp.broadcast_to(dr & 15, (LANES,))]
                flv = jnp.where(lane == 0, fl, 0.0)
                keep = (flv[0] * m1[r]) > 0

                @pl.when(keep)
                def _():
                    off2 = (cnt >> 4) * LANES
                    sel = lane == (cnt & 15)
                    rb = csrc[pl.ds(off2, LANES)]
                    csrc[pl.ds(off2, LANES)] = jnp.where(sel, si[r], rb)
                    rb2 = cdst[pl.ds(off2, LANES)]
                    cdst[pl.ds(off2, LANES)] = jnp.where(sel, dr, rb2)
                    rb3 = cw[pl.ds(off2, LANES)]
                    cw[pl.ds(off2, LANES)] = jnp.where(sel, wc[r], rb3)

                cnt = cnt + jnp.where(keep, 1, 0)
            return cnt

        cnt = lax.fori_loop(0, NCHUNK, cb, 0)

        nch = (cnt + 15) >> 4
        nch = ((nch + NBUF - 1) >> 2) << 2
        target = nch * LANES + NBUF * LANES

        def padb(k, cnt2):
            off2 = (cnt2 >> 4) * LANES
            sel = lane == (cnt2 & 15)
            rb = csrc[pl.ds(off2, LANES)]
            csrc[pl.ds(off2, LANES)] = jnp.where(sel, 0, rb)
            rb2 = cdst[pl.ds(off2, LANES)]
            cdst[pl.ds(off2, LANES)] = jnp.where(sel, NPAD - 1, rb2)
            rb3 = cw[pl.ds(off2, LANES)]
            cw[pl.ds(off2, LANES)] = jnp.where(sel, 0.0, rb3)
            return cnt2 + 1

        lax.fori_loop(0, target - cnt, padb, cnt)
        src_e, dst_e, w_e = csrc, cdst, cw
    else:
        src_e, dst_e, w_e = src_v, dst_v, w_v

    def fire_gather(i, b):
        if nfilt:
            gi = src_e[pl.ds(i * LANES, LANES)]
        else:
            m = (i * LANES + lane) < EPT
            gi = jnp.where(m, src_e[pl.ds(i * LANES, LANES)], 0)
        idx_g[b, :] = gi
        pltpu.async_copy(tab_hbm.at[idx_g.at[b]], rows.at[b], gsem[b])

    def wait_gather(b):
        pltpu.make_async_copy(tab_hbm.at[idx_g.at[b]], rows.at[b],
                              gsem[b]).wait()

    def fire_scatter(i, b):
        if nfilt:
            di = dst_e[pl.ds(i * LANES, LANES)]
            wch = w_e[pl.ds(i * LANES, LANES)]
        else:
            m = (i * LANES + lane) < EPT
            di = jnp.where(m, dst_e[pl.ds(i * LANES, LANES)], NPAD - 1)
            wch = jnp.where(m, w_e[pl.ds(i * LANES, LANES)], 0.0)
        idx_s[b, :] = di
        for r in range(LANES):
            wr = wch[r]
            for cc in range(D // LANES):
                sl = pl.ds(cc * LANES, LANES)
                rows[b, r, sl] = rows[b, r, sl] * wr
        pltpu.async_copy(rows.at[b], out_hbm.at[c].at[idx_s.at[b]], ssem[b],
                         add=True)

    def wait_scatter(b):
        pltpu.make_async_copy(rows.at[b], out_hbm.at[c].at[idx_s.at[b]],
                              ssem[b]).wait()

    for b in range(NBUF):
        fire_gather(b, b)

    def group(g, _):
        for b in range(NBUF):
            wait_gather(b)
            fire_scatter(g * NBUF + b, b)
        for b in range(NBUF):
            wait_scatter(b)
            fire_gather(g * NBUF + NBUF + b, b)
        return 0

    if nfilt:
        lax.fori_loop(0, nch >> 2, group, 0)
        for b in range(NBUF):
            wait_gather(b)
    else:
        ngroup = (NCHUNK - 1) // NBUF
        lax.fori_loop(0, ngroup, group, 0)
        wait_gather(0)
        fire_scatter(NCHUNK - 1, 0)
        wait_scatter(0)
        for b in range(1, NBUF):
            wait_gather(b)


def _row_pass(src, dst, w, table, sel=None):
    pad_e = (NCHUNK + NBUF) * LANES
    nfilt = 0 if sel is None else sel.shape[0]
    scratch = [
        pltpu.VMEM((pad_e,), jnp.int32),
        pltpu.VMEM((pad_e,), jnp.int32),
        pltpu.VMEM((pad_e,), jnp.float32),
        pltpu.VMEM((NBUF, LANES), jnp.int32),
        pltpu.VMEM((NBUF, LANES), jnp.int32),
        pltpu.VMEM((NBUF, LANES, D), jnp.float32),
        pltpu.VMEM((LANES, D), jnp.float32),
    ]
    if nfilt:
        scratch += [
            pltpu.VMEM((nfilt,), jnp.int32),
            pltpu.VMEM((NPAD,), jnp.float32),
            pltpu.VMEM((pad_e + 2 * NBUF * LANES,), jnp.int32),
            pltpu.VMEM((pad_e + 2 * NBUF * LANES,), jnp.int32),
            pltpu.VMEM((pad_e + 2 * NBUF * LANES,), jnp.float32),
        ]
    scratch += [pltpu.SemaphoreType.DMA] * (2 * NBUF)
    fn = pl.kernel(
        functools.partial(_row_pass_body, nfilt),
        out_type=jax.ShapeDtypeStruct((NSC, NPAD, D), jnp.float32),
        mesh=_mesh,
        scratch_types=scratch,
    )
    if nfilt:
        return fn(src, dst, w, table, sel)
    return fn(src, dst, w, table)


def _combine_body(p0_ref, p1_ref, dp_ref, base_ref, out_ref):
    dsum = jnp.sum(dp_ref[...], axis=0)
    scale = jnp.where(dsum > 0, 0.1 / dsum, 0.0)[:, None]
    out_ref[...] = (p0_ref[...] + p1_ref[...]) * scale + base_ref[...]


def _gat_combine(parts, dpart, base):
    bm = 1024
    return pl.pallas_call(
        _combine_body,
        grid=(NPAD // bm,),
        in_specs=[
            pl.BlockSpec((bm, D), lambda i: (i, 0)),
            pl.BlockSpec((bm, D), lambda i: (i, 0)),
            pl.BlockSpec((NSC * NTILE, bm), lambda i: (0, i)),
            pl.BlockSpec((bm, D), lambda i: (i, 0)),
        ],
        out_specs=pl.BlockSpec((bm, D), lambda i: (i, 0)),
        out_shape=jax.ShapeDtypeStruct((NPAD, D), jnp.float32),
    )(parts[0], parts[1], dpart, base)


def _batch_gather_body(pg_hbm, pd_hbm, uids_hbm, pos_hbm, neg_hbm,
                       u0_hbm, u1_hbm, p0_hbm, p1_hbm, n0_hbm, n1_hbm,
                       idx_v, rows, sem):
    c = lax.axis_index("c")
    s = lax.axis_index("s")
    wid = c * NTILE + s
    bpt = BATCH // (NSC * NTILE)
    base = wid * bpt
    for ids_hbm, tab_hbm, out_hbm, half in (
            (uids_hbm, pg_hbm, u0_hbm, 0), (uids_hbm, pg_hbm, u1_hbm, 1),
            (pos_hbm, pd_hbm, p0_hbm, 0), (pos_hbm, pd_hbm, p1_hbm, 1),
            (neg_hbm, pd_hbm, n0_hbm, 0), (neg_hbm, pd_hbm, n1_hbm, 1)):
        pltpu.sync_copy(ids_hbm.at[pl.ds(base, bpt)], idx_v)
        pltpu.async_copy(tab_hbm.at[half].at[idx_v], rows, sem).wait()
        pltpu.sync_copy(rows, out_hbm.at[pl.ds(base, bpt)])


def _batch_gather(pg, pd, uids, pos, neg):
    bpt = BATCH // (NSC * NTILE)
    return pl.kernel(
        _batch_gather_body,
        out_type=[jax.ShapeDtypeStruct((BATCH, D), jnp.float32)] * 6,
        mesh=_mesh,
        scratch_types=[
            pltpu.VMEM((bpt,), jnp.int32),
            pltpu.VMEM((bpt, D), jnp.float32),
            pltpu.SemaphoreType.DMA,
        ],
    )(pg, pd, uids, pos, neg)


def _softplus(x):
    return jnp.maximum(x, 0.0) + jnp.log(1.0 + jnp.exp(-jnp.abs(x)))


def _mlp_body(u0_ref, u1_ref, p0_ref, p1_ref, n0_ref, n1_ref,
              w1a_ref, w1b_ref, b1_ref, w2_ref, b2_ref, w3_ref, b3_ref,
              *rest):
    small_refs = rest[:-4]
    t1_ref, t2_ref, t3_ref, reg_ref = rest[-4:]
    i = pl.program_id(0)
    u = u0_ref[...] + u1_ref[...]
    uw = jnp.dot(u, w1a_ref[...], preferred_element_type=jnp.float32)

    def score(x):
        h = jnp.maximum(
            uw + jnp.dot(x, w1b_ref[...],
                         preferred_element_type=jnp.float32) + b1_ref[...],
            0.0)
        h = jnp.maximum(
            jnp.dot(h, w2_ref[...], preferred_element_type=jnp.float32)
            + b2_ref[...], 0.0)
        return (jnp.dot(h, w3_ref[...], preferred_element_type=jnp.float32)
                + b3_ref[...])[:, 0]

    sp = score(p0_ref[...] + p1_ref[...])
    sn = score(n0_ref[...] + n1_ref[...])

    @pl.when(i == 0)
    def _():
        t1_ref[...] = jnp.zeros_like(t1_ref)
        t2_ref[...] = jnp.zeros_like(t2_ref)
        t3_ref[...] = jnp.zeros_like(t3_ref)
        rs = jnp.float32(0.0)
        for ref in (w1a_ref, w1b_ref, b1_ref, w2_ref, b2_ref, w3_ref, b3_ref):
            rs += jnp.sum(ref[...] * ref[...])
        for ref in small_refs:
            rs += jnp.sum(ref[...] * ref[...])
        reg_ref[...] = jnp.full_like(reg_ref, rs)

    t1_ref[...] += jnp.sum(_softplus(-sp))
    t2_ref[...] += jnp.sum(_softplus(sn))
    t3_ref[...] += jnp.sum(_softplus(sn - sp))


def _mlp_loss(embs, w1a, w1b, b1, w2, b2, w3, b3, smalls):
    bm = 512
    full = lambda shape: pl.BlockSpec(shape, lambda i: tuple(0 for _ in shape))
    in_specs = (
        [pl.BlockSpec((bm, D), lambda i: (i, 0))] * 6
        + [full((D, D)), full((D, D)), full((1, D)), full((D, D)),
           full((1, D)), full((D, 1)), full((1, 1))]
        + [full(x.shape) for x in smalls]
    )
    return pl.pallas_call(
        _mlp_body,
        grid=(BATCH // bm,),
        in_specs=in_specs,
        out_specs=[full((1, 1))] * 4,
        out_shape=[jax.ShapeDtypeStruct((1, 1), jnp.float32)] * 4,
    )(*embs, w1a, w1b, b1, w2, b2, w3, b3, *smalls)


def kernel(E_g_0, E_d_0, adj_rows, adj_cols, adj_vals, gg_edges, dd_edges,
           uids, iids, pos, neg, att2_W, att2_a, att1_W, att1_a, m_W1, m_b1,
           m_W2, m_b2, m_W3, m_b3, m1_W1, m1_b1, m1_W2, m1_b2):
    wh_d, s1_d, s2_d, ssq_d = _dense_gat(E_d_0, att2_W, att2_a)
    wh_g, s1_g, s2_g, ssq_g = _dense_gat(E_g_0, att2_W, att2_a)

    dd_src, dd_dst = dd_edges[0], dd_edges[1]
    gg_src, gg_dst = gg_edges[0], gg_edges[1]
    ee_d, dpart_d = _edge_scores(dd_src, dd_dst, s1_d[:, 0], s2_d[:, 0])
    ee_g, dpart_g = _edge_scores(gg_src, gg_dst, s1_g[:, 0], s2_g[:, 0])

    agg_d = _row_pass(dd_src, dd_dst, ee_d, wh_d)
    agg_g = _row_pass(gg_src, gg_dst, ee_g, wh_g)
    E_d0 = _gat_combine(agg_d, dpart_d, E_d_0)
    E_g0 = _gat_combine(agg_g, dpart_g, E_g_0)

    pg = _row_pass(adj_cols, adj_rows, adj_vals, E_d0)
    pd = _row_pass(adj_rows, adj_cols, adj_vals, E_g0)

    embs = _batch_gather(pg, pd, uids, pos, neg)

    smalls = [att2_W, att2_a.reshape(4, 128), att1_W, att1_a.reshape(4, 128),
              m1_W1, m1_b1.reshape(2, 128), m1_W2, m1_b2.reshape(2, 128)]
    t1, t2, t3, reg_small = _mlp_loss(
        embs, m_W1[:D], m_W1[D:], m_b1.reshape(1, D),
        m_W2, m_b2.reshape(1, D), m_W3, m_b3.reshape(1, 1), smalls)

    loss_r = (t1[0, 0] + t2[0, 0] + t3[0, 0]) / BATCH
    loss_reg = LAMBDA_2 * (ssq_d[0, 0] + ssq_g[0, 0] + reg_small[0, 0])
    loss = loss_reg + loss_r
    return (loss, loss_r, jnp.float32(0.0))


# speedup vs baseline: 1.3095x; 1.3095x over previous
import functools

import jax
import jax.numpy as jnp
from jax import lax
from jax.experimental import pallas as pl
from jax.experimental.pallas import tpu as pltpu
from jax.experimental.pallas import tpu_sc as plsc

N = 10000
NPAD = 10240
D = 256
E = 160000
BATCH = 4096
NSC = 2
NTILE = 16
LANES = 16
EPT = E // (NSC * NTILE)
NCHUNK = (EPT + LANES - 1) // LANES
ALPHA = 0.2
LAMBDA_2 = 1e-4

_mesh = plsc.VectorSubcoreMesh(core_axis_name="c", subcore_axis_name="s")


def _dense_gat_body(h_ref, w_ref, a1_ref, a2_ref, wh_ref, s1_ref, s2_ref,
                    ssq_ref):
    i = pl.program_id(0)
    h = h_ref[...]
    wh = jnp.dot(h, w_ref[...], preferred_element_type=jnp.float32)
    wh_ref[...] = wh
    s1_ref[...] = jnp.dot(wh, a1_ref[...], preferred_element_type=jnp.float32)
    s2_ref[...] = jnp.dot(wh, a2_ref[...], preferred_element_type=jnp.float32)

    @pl.when(i == 0)
    def _():
        ssq_ref[...] = jnp.zeros_like(ssq_ref)

    ssq_ref[...] += jnp.sum(h * h)


def _dense_gat(h, w, a):
    bm = 1000
    a1 = a[:D][:, None]
    a2 = a[D:][:, None]
    return pl.pallas_call(
        _dense_gat_body,
        grid=(N // bm,),
        in_specs=[
            pl.BlockSpec((bm, D), lambda i: (i, 0)),
            pl.BlockSpec((D, D), lambda i: (0, 0)),
            pl.BlockSpec((D, 1), lambda i: (0, 0)),
            pl.BlockSpec((D, 1), lambda i: (0, 0)),
        ],
        out_specs=[
            pl.BlockSpec((bm, D), lambda i: (i, 0)),
            pl.BlockSpec((bm, 1), lambda i: (i, 0)),
            pl.BlockSpec((bm, 1), lambda i: (i, 0)),
            pl.BlockSpec((1, 1), lambda i: (0, 0)),
        ],
        out_shape=[
            jax.ShapeDtypeStruct((N, D), jnp.float32),
            jax.ShapeDtypeStruct((N, 1), jnp.float32),
            jax.ShapeDtypeStruct((N, 1), jnp.float32),
            jax.ShapeDtypeStruct((1, 1), jnp.float32),
        ],
    )(h, w, a1, a2)


def _edge_score_body(src_hbm, dst_hbm, s1_hbm, s2_hbm, eexp_hbm, dpart_hbm,
                     s1_tab, s2_tab, src_v, dst_v, ee_v, dloc):
    c = lax.axis_index("c")
    s = lax.axis_index("s")
    wid = c * NTILE + s
    base = wid * EPT
    pltpu.sync_copy(s1_hbm, s1_tab)
    pltpu.sync_copy(s2_hbm, s2_tab)
    pltpu.sync_copy(src_hbm.at[pl.ds(base, EPT)], src_v.at[pl.ds(0, EPT)])
    pltpu.sync_copy(dst_hbm.at[pl.ds(base, EPT)], dst_v.at[pl.ds(0, EPT)])

    lane = lax.iota(jnp.int32, LANES)
    zv = jnp.zeros((LANES,), jnp.float32)

    def zero_body(i, _):
        dloc[pl.ds(i * LANES, LANES)] = zv
        return 0

    lax.fori_loop(0, NPAD // LANES, zero_body, 0)

    def body(i, _):
        m = (i * LANES + lane) < EPT
        si = jnp.where(m, src_v[pl.ds(i * LANES, LANES)], 0)
        di = jnp.where(m, dst_v[pl.ds(i * LANES, LANES)], 0)
        ev = zv
        for r in range(LANES):
            sr = si[r]
            dr = di[r]
            row1 = s1_tab[pl.ds((sr >> 4) * LANES, LANES)]
            v1 = row1[jnp.broadcast_to(sr & 15, (LANES,))]
            row2 = s2_tab[pl.ds((dr >> 4) * LANES, LANES)]
            v2 = row2[jnp.broadcast_to(dr & 15, (LANES,))]
            ev = ev + jnp.where(lane == r, v1 + v2, 0.0)
        ev = jnp.where(ev > 0, ev, ALPHA * ev)
        ev = jnp.exp(ev)
        ee_v[pl.ds(i * LANES, LANES)] = ev
        eem = jnp.where(m, ev, 0.0)
        for r in range(LANES):
            dr = di[r]
            off = (dr >> 4) * LANES
            rowd = dloc[pl.ds(off, LANES)]
            dloc[pl.ds(off, LANES)] = rowd + jnp.where(lane == (dr & 15),
                                                       eem[r], 0.0)
        return 0

    lax.fori_loop(0, NCHUNK, body, 0)
    pltpu.sync_copy(ee_v.at[pl.ds(0, EPT)], eexp_hbm.at[pl.ds(base, EPT)])
    pltpu.sync_copy(dloc, dpart_hbm.at[wid])


def _edge_scores(src, dst, s1, s2):
    pad_e = NCHUNK * LANES
    return pl.kernel(
        _edge_score_body,
        out_type=[
            jax.ShapeDtypeStruct((E,), jnp.float32),
            jax.ShapeDtypeStruct((NSC * NTILE, NPAD), jnp.float32),
        ],
        mesh=_mesh,
        scratch_types=[
            pltpu.VMEM((N,), jnp.float32),
            pltpu.VMEM((N,), jnp.float32),
            pltpu.VMEM((pad_e,), jnp.int32),
            pltpu.VMEM((pad_e,), jnp.int32),
            pltpu.VMEM((pad_e,), jnp.float32),
            pltpu.VMEM((NPAD,), jnp.float32),
        ],
    )(src, dst, s1, s2)


NBUF = 4


def _row_pass_body(nfilt, src_hbm, dst_hbm, w_hbm, tab_hbm, *rest):
    if nfilt:
        ids_hbm, out_hbm = rest[0], rest[1]
        (src_v, dst_v, w_v, idx_g, idx_s, rows, zb,
         ids_v, flag, csrc, cdst, cw) = rest[2:14]
        sems = rest[14:]
    else:
        out_hbm = rest[0]
        (src_v, dst_v, w_v, idx_g, idx_s, rows, zb) = rest[1:8]
        sems = rest[8:]
    gsem = sems[:NBUF]
    ssem = sems[NBUF:]
    c = lax.axis_index("c")
    s = lax.axis_index("s")
    wid = c * NTILE + s
    base = wid * EPT

    z = jnp.zeros((LANES,), jnp.float32)
    for r in range(LANES):
        for cc in range(D // LANES):
            zb[r, pl.ds(cc * LANES, LANES)] = z

    nz = NPAD // NTILE // LANES

    def zloop(j, _):
        pltpu.sync_copy(zb, out_hbm.at[c].at[pl.ds((s * nz + j) * LANES,
                                                   LANES)])
        return 0

    lax.fori_loop(0, nz, zloop, 0)
    plsc.subcore_barrier()

    pltpu.sync_copy(src_hbm.at[pl.ds(base, EPT)], src_v.at[pl.ds(0, EPT)])
    pltpu.sync_copy(dst_hbm.at[pl.ds(base, EPT)], dst_v.at[pl.ds(0, EPT)])
    pltpu.sync_copy(w_hbm.at[pl.ds(base, EPT)], w_v.at[pl.ds(0, EPT)])

    lane = lax.iota(jnp.int32, LANES)

    if nfilt:
        pltpu.sync_copy(ids_hbm, ids_v)
        zv = jnp.zeros((LANES,), jnp.float32)

        def zf(j, _):
            flag[pl.ds(j * LANES, LANES)] = zv
            return 0

        lax.fori_loop(0, NPAD // LANES, zf, 0)

        def fb(j, _):
            idc = ids_v[pl.ds(j * LANES, LANES)]
            for r in range(LANES):
                idr = idc[r]
                off = (idr >> 4) * LANES
                rowf = flag[pl.ds(off, LANES)]
                flag[pl.ds(off, LANES)] = jnp.where(lane == (idr & 15),
                                                    1.0, rowf)
            return 0

        lax.fori_loop(0, nfilt // LANES, fb, 0)

        def cb(j, cnt):
            m1 = jnp.where((j * LANES + lane) < EPT, 1.0, 0.0)
            si = src_v[pl.ds(j * LANES, LANES)]
            di = jnp.where(m1 > 0, dst_v[pl.ds(j * LANES, LANES)], 0)
            wc = w_v[pl.ds(j * LANES, LANES)]
            for r in range(LANES):
                dr = di[r]
                flrow = flag[pl.ds((dr >> 4) * LANES, LANES)]
                fl = flrow[jnp.broadcast_to(dr & 15, (LANES,))]
                flv = jnp.where(lane == 0, fl, 0.0)
                keep = (flv[0] * m1[r]) > 0

                @pl.when(keep)
                def _():
                    off2 = (cnt >> 4) * LANES
                    sel = lane == (cnt & 15)
                    rb = csrc[pl.ds(off2, LANES)]
                    csrc[pl.ds(off2, LANES)] = jnp.where(sel, si[r], rb)
                    rb2 = cdst[pl.ds(off2, LANES)]
                    cdst[pl.ds(off2, LANES)] = jnp.where(sel, dr, rb2)
                    rb3 = cw[pl.ds(off2, LANES)]
                    cw[pl.ds(off2, LANES)] = jnp.where(sel, wc[r], rb3)

                cnt = cnt + jnp.where(keep, 1, 0)
            return cnt

        cnt = lax.fori_loop(0, NCHUNK, cb, 0)

        nch = (cnt + 15) >> 4
        nch = ((nch + NBUF - 1) >> 2) << 2
        target = nch * LANES + NBUF * LANES

        def padb(k, cnt2):
            off2 = (cnt2 >> 4) * LANES
            sel = lane == (cnt2 & 15)
            rb = csrc[pl.ds(off2, LANES)]
            csrc[pl.ds(off2, LANES)] = jnp.where(sel, 0, rb)
            rb2 = cdst[pl.ds(off2, LANES)]
            cdst[pl.ds(off2, LANES)] = jnp.where(sel, NPAD - 1, rb2)
            rb3 = cw[pl.ds(off2, LANES)]
            cw[pl.ds(off2, LANES)] = jnp.where(sel, 0.0, rb3)
            return cnt2 + 1

        lax.fori_loop(0, target - cnt, padb, cnt)
        src_e, dst_e, w_e = csrc, cdst, cw
    else:
        src_e, dst_e, w_e = src_v, dst_v, w_v

    def fire_gather(i, b):
        if nfilt:
            gi = src_e[pl.ds(i * LANES, LANES)]
        else:
            m = (i * LANES + lane) < EPT
            gi = jnp.where(m, src_e[pl.ds(i * LANES, LANES)], 0)
        idx_g[b, :] = gi
        pltpu.async_copy(tab_hbm.at[idx_g.at[b]], rows.at[b], gsem[b])

    def wait_gather(b):
        pltpu.make_async_copy(tab_hbm.at[idx_g.at[b]], rows.at[b],
                              gsem[b]).wait()

    def fire_scatter(i, b):
        if nfilt:
            di = dst_e[pl.ds(i * LANES, LANES)]
            wch = w_e[pl.ds(i * LANES, LANES)]
        else:
            m = (i * LANES + lane) < EPT
            di = jnp.where(m, dst_e[pl.ds(i * LANES, LANES)], NPAD - 1)
            wch = jnp.where(m, w_e[pl.ds(i * LANES, LANES)], 0.0)
        idx_s[b, :] = di
        for r in range(LANES):
            wr = wch[r]
            for cc in range(D // LANES):
                sl = pl.ds(cc * LANES, LANES)
                rows[b, r, sl] = rows[b, r, sl] * wr
        pltpu.async_copy(rows.at[b], out_hbm.at[c].at[idx_s.at[b]], ssem[b],
                         add=True)

    def wait_scatter(b):
        pltpu.make_async_copy(rows.at[b], out_hbm.at[c].at[idx_s.at[b]],
                              ssem[b]).wait()

    for b in range(NBUF):
        fire_gather(b, b)

    def group(g, _):
        for b in range(NBUF):
            wait_gather(b)
            fire_scatter(g * NBUF + b, b)
        for b in range(NBUF):
            wait_scatter(b)
            fire_gather(g * NBUF + NBUF + b, b)
        return 0

    if nfilt:
        lax.fori_loop(0, nch >> 2, group, 0)
        for b in range(NBUF):
            wait_gather(b)
    else:
        ngroup = (NCHUNK - 1) // NBUF
        lax.fori_loop(0, ngroup, group, 0)
        wait_gather(0)
        fire_scatter(NCHUNK - 1, 0)
        wait_scatter(0)
        for b in range(1, NBUF):
            wait_gather(b)


def _row_pass(src, dst, w, table, sel=None):
    pad_e = (NCHUNK + NBUF) * LANES
    nfilt = 0 if sel is None else sel.shape[0]
    scratch = [
        pltpu.VMEM((pad_e,), jnp.int32),
        pltpu.VMEM((pad_e,), jnp.int32),
        pltpu.VMEM((pad_e,), jnp.float32),
        pltpu.VMEM((NBUF, LANES), jnp.int32),
        pltpu.VMEM((NBUF, LANES), jnp.int32),
        pltpu.VMEM((NBUF, LANES, D), jnp.float32),
        pltpu.VMEM((LANES, D), jnp.float32),
    ]
    if nfilt:
        scratch += [
            pltpu.VMEM((nfilt,), jnp.int32),
            pltpu.VMEM((NPAD,), jnp.float32),
            pltpu.VMEM((pad_e + 2 * NBUF * LANES,), jnp.int32),
            pltpu.VMEM((pad_e + 2 * NBUF * LANES,), jnp.int32),
            pltpu.VMEM((pad_e + 2 * NBUF * LANES,), jnp.float32),
        ]
    scratch += [pltpu.SemaphoreType.DMA] * (2 * NBUF)
    fn = pl.kernel(
        functools.partial(_row_pass_body, nfilt),
        out_type=jax.ShapeDtypeStruct((NSC, NPAD, D), jnp.float32),
        mesh=_mesh,
        scratch_types=scratch,
    )
    if nfilt:
        return fn(src, dst, w, table, sel)
    return fn(src, dst, w, table)


def _combine_body(p0_ref, p1_ref, dp_ref, base_ref, out_ref):
    dsum = jnp.sum(dp_ref[...], axis=0)
    scale = jnp.where(dsum > 0, 0.1 / dsum, 0.0)[:, None]
    out_ref[...] = (p0_ref[...] + p1_ref[...]) * scale + base_ref[...]


def _gat_combine(parts, dpart, base):
    bm = 1024
    return pl.pallas_call(
        _combine_body,
        grid=(NPAD // bm,),
        in_specs=[
            pl.BlockSpec((bm, D), lambda i: (i, 0)),
            pl.BlockSpec((bm, D), lambda i: (i, 0)),
            pl.BlockSpec((NSC * NTILE, bm), lambda i: (0, i)),
            pl.BlockSpec((bm, D), lambda i: (i, 0)),
        ],
        out_specs=pl.BlockSpec((bm, D), lambda i: (i, 0)),
        out_shape=jax.ShapeDtypeStruct((NPAD, D), jnp.float32),
    )(parts[0], parts[1], dpart, base)


def _batch_gather_body(pg_hbm, pd_hbm, uids_hbm, pos_hbm, neg_hbm,
                       u0_hbm, u1_hbm, p0_hbm, p1_hbm, n0_hbm, n1_hbm,
                       idx_v, rows, sem):
    c = lax.axis_index("c")
    s = lax.axis_index("s")
    wid = c * NTILE + s
    bpt = BATCH // (NSC * NTILE)
    base = wid * bpt
    for ids_hbm, tab_hbm, out_hbm, half in (
            (uids_hbm, pg_hbm, u0_hbm, 0), (uids_hbm, pg_hbm, u1_hbm, 1),
            (pos_hbm, pd_hbm, p0_hbm, 0), (pos_hbm, pd_hbm, p1_hbm, 1),
            (neg_hbm, pd_hbm, n0_hbm, 0), (neg_hbm, pd_hbm, n1_hbm, 1)):
        pltpu.sync_copy(ids_hbm.at[pl.ds(base, bpt)], idx_v)
        pltpu.async_copy(tab_hbm.at[half].at[idx_v], rows, sem).wait()
        pltpu.sync_copy(rows, out_hbm.at[pl.ds(base, bpt)])


def _batch_gather(pg, pd, uids, pos, neg):
    bpt = BATCH // (NSC * NTILE)
    return pl.kernel(
        _batch_gather_body,
        out_type=[jax.ShapeDtypeStruct((BATCH, D), jnp.float32)] * 6,
        mesh=_mesh,
        scratch_types=[
            pltpu.VMEM((bpt,), jnp.int32),
            pltpu.VMEM((bpt, D), jnp.float32),
            pltpu.SemaphoreType.DMA,
        ],
    )(pg, pd, uids, pos, neg)


def _softplus(x):
    return jnp.maximum(x, 0.0) + jnp.log(1.0 + jnp.exp(-jnp.abs(x)))


def _mlp_body(u0_ref, u1_ref, p0_ref, p1_ref, n0_ref, n1_ref,
              w1a_ref, w1b_ref, b1_ref, w2_ref, b2_ref, w3_ref, b3_ref,
              *rest):
    small_refs = rest[:-4]
    t1_ref, t2_ref, t3_ref, reg_ref = rest[-4:]
    i = pl.program_id(0)
    u = u0_ref[...] + u1_ref[...]
    uw = jnp.dot(u, w1a_ref[...], preferred_element_type=jnp.float32)

    def score(x):
        h = jnp.maximum(
            uw + jnp.dot(x, w1b_ref[...],
                         preferred_element_type=jnp.float32) + b1_ref[...],
            0.0)
        h = jnp.maximum(
            jnp.dot(h, w2_ref[...], preferred_element_type=jnp.float32)
            + b2_ref[...], 0.0)
        return (jnp.dot(h, w3_ref[...], preferred_element_type=jnp.float32)
                + b3_ref[...])[:, 0]

    sp = score(p0_ref[...] + p1_ref[...])
    sn = score(n0_ref[...] + n1_ref[...])

    @pl.when(i == 0)
    def _():
        t1_ref[...] = jnp.zeros_like(t1_ref)
        t2_ref[...] = jnp.zeros_like(t2_ref)
        t3_ref[...] = jnp.zeros_like(t3_ref)
        rs = jnp.float32(0.0)
        for ref in (w1a_ref, w1b_ref, b1_ref, w2_ref, b2_ref, w3_ref, b3_ref):
            rs += jnp.sum(ref[...] * ref[...])
        for ref in small_refs:
            rs += jnp.sum(ref[...] * ref[...])
        reg_ref[...] = jnp.full_like(reg_ref, rs)

    t1_ref[...] += jnp.sum(_softplus(-sp))
    t2_ref[...] += jnp.sum(_softplus(sn))
    t3_ref[...] += jnp.sum(_softplus(sn - sp))


def _mlp_loss(embs, w1a, w1b, b1, w2, b2, w3, b3, smalls):
    bm = 512
    full = lambda shape: pl.BlockSpec(shape, lambda i: tuple(0 for _ in shape))
    in_specs = (
        [pl.BlockSpec((bm, D), lambda i: (i, 0))] * 6
        + [full((D, D)), full((D, D)), full((1, D)), full((D, D)),
           full((1, D)), full((D, 1)), full((1, 1))]
        + [full(x.shape) for x in smalls]
    )
    return pl.pallas_call(
        _mlp_body,
        grid=(BATCH // bm,),
        in_specs=in_specs,
        out_specs=[full((1, 1))] * 4,
        out_shape=[jax.ShapeDtypeStruct((1, 1), jnp.float32)] * 4,
    )(*embs, w1a, w1b, b1, w2, b2, w3, b3, *smalls)


def kernel(E_g_0, E_d_0, adj_rows, adj_cols, adj_vals, gg_edges, dd_edges,
           uids, iids, pos, neg, att2_W, att2_a, att1_W, att1_a, m_W1, m_b1,
           m_W2, m_b2, m_W3, m_b3, m1_W1, m1_b1, m1_W2, m1_b2):
    wh_d, s1_d, s2_d, ssq_d = _dense_gat(E_d_0, att2_W, att2_a)
    wh_g, s1_g, s2_g, ssq_g = _dense_gat(E_g_0, att2_W, att2_a)

    dd_src, dd_dst = dd_edges[0], dd_edges[1]
    gg_src, gg_dst = gg_edges[0], gg_edges[1]
    ee_d, dpart_d = _edge_scores(dd_src, dd_dst, s1_d[:, 0], s2_d[:, 0])
    ee_g, dpart_g = _edge_scores(gg_src, gg_dst, s1_g[:, 0], s2_g[:, 0])

    agg_d = _row_pass(dd_src, dd_dst, ee_d, wh_d)
    agg_g = _row_pass(gg_src, gg_dst, ee_g, wh_g)
    E_d0 = _gat_combine(agg_d, dpart_d, E_d_0)
    E_g0 = _gat_combine(agg_g, dpart_g, E_g_0)

    pg = _row_pass(adj_cols, adj_rows, adj_vals, E_d0)
    pd = _row_pass(adj_rows, adj_cols, adj_vals, E_g0)

    embs = _batch_gather(pg, pd, uids, pos, neg)

    smalls = [att2_W, att2_a.reshape(4, 128), att1_W, att1_a.reshape(4, 128),
              m1_W1, m1_b1.reshape(2, 128), m1_W2, m1_b2.reshape(2, 128)]
    t1, t2, t3, reg_small = _mlp_loss(
        embs, m_W1[:D], m_W1[D:], m_b1.reshape(1, D),
        m_W2, m_b2.reshape(1, D), m_W3, m_b3.reshape(1, 1), smalls)

    loss_r = (t1[0, 0] + t2[0, 0] + t3[0, 0]) / BATCH
    loss_reg = LAMBDA_2 * (ssq_d[0, 0] + ssq_g[0, 0] + reg_small[0, 0])
    loss = loss_reg + loss_r
    return (loss, loss_r, jnp.float32(0.0))
